# baseline (speedup 1.0000x reference)
import functools

import jax
import jax.numpy as jnp
from jax.experimental import pallas as pl
from jax.experimental.pallas import tpu as pltpu
from jax.experimental.pallas import tpu_sc as plsc

_EPS = 1e-5
_NW = 32
_IDXW = 128
_CH = 2
_CE = _CH * _IDXW
_LANES = 128


def _bn_body(x_ref, g_ref, b_ref, z_ref):
    x = x_ref[...]
    n = x.shape[0]
    mu = jnp.sum(x, axis=0, keepdims=True) / n
    xc = x - mu
    var = jnp.sum(xc * xc, axis=0, keepdims=True) / n
    z_ref[...] = g_ref[...] * xc * jax.lax.rsqrt(var + _EPS) + b_ref[...]


def _batchnorm(x, gamma, beta):
    n, d = x.shape
    return pl.pallas_call(
        _bn_body,
        out_shape=jax.ShapeDtypeStruct((n, d), jnp.float32),
    )(x, gamma.reshape(1, d), beta.reshape(1, d))


def _mlp_body(xi_ref, xj_ref, w1_ref, b1_ref, w2_ref, b2_ref, w3_ref, b3_ref,
              o_ref, *, last_relu):
    din = w1_ref.shape[0] // 2
    dout = w3_ref.shape[1]
    be = xi_ref.shape[0]

    def mm(a, w):
        if a.dtype != jnp.bfloat16:
            a = a.astype(jnp.bfloat16)
        return jnp.dot(a, w, preferred_element_type=jnp.float32)

    h = jnp.concatenate([xi_ref[...][:, :din], xj_ref[...][:, :din]], axis=1)
    h = jnp.maximum(mm(h, w1_ref[...]) + b1_ref[...], 0.0)
    h = jnp.maximum(mm(h, w2_ref[...]) + b2_ref[...], 0.0)
    o = mm(h, w3_ref[...]) + b3_ref[...]
    if last_relu:
        o = jnp.maximum(o, 0.0)
    if dout < _LANES:
        o = jnp.concatenate(
            [o, jnp.full((be, 8), 1.0, jnp.float32),
             jnp.zeros((be, _LANES - dout - 8), jnp.float32)], axis=1)
    o_ref[...] = o


def _edge_mlp(xi, xj, w1, b1, w2, b2, w3, b3, last_relu, be=2048):
    e, din = xi.shape
    big = w1.shape[1]
    dout = w3.shape[1]
    w1 = w1.astype(jnp.bfloat16)
    w2 = w2.astype(jnp.bfloat16)
    w3 = w3.astype(jnp.bfloat16)
    b1 = b1.reshape(1, big)
    b2 = b2.reshape(1, big)
    b3 = b3.reshape(1, dout)
    grid = e // be
    wspec = lambda a: pl.BlockSpec(a.shape, lambda i: (0, 0))
    return pl.pallas_call(
        functools.partial(_mlp_body, last_relu=last_relu),
        grid=(grid,),
        in_specs=[
            pl.BlockSpec((be, din), lambda i: (i, 0)),
            pl.BlockSpec((be, din), lambda i: (i, 0)),
            wspec(w1), wspec(b1), wspec(w2), wspec(b2), wspec(w3), wspec(b3),
        ],
        out_specs=pl.BlockSpec((be, _LANES), lambda i: (i, 0)),
        out_shape=jax.ShapeDtypeStruct((e, _LANES), jnp.float32),
        compiler_params=pltpu.CompilerParams(
            dimension_semantics=("arbitrary",)),
    )(xi, xj, w1, b1, w2, b2, w3, b3)


def _combine_body(*refs, keep, nparts):
    y_ref = refs[-1]
    psum = refs[0][...][0] + refs[0][...][1]
    for r in refs[1:nparts]:
        psum = psum + r[...][0] + r[...][1]
    c = refs[nparts][...][0] + refs[nparts][...][1]
    for r in refs[nparts + 1:2 * nparts]:
        c = c + r[...][0] + r[...][1]
    cnt = c[:, 64:65]
    y = psum / jnp.maximum(cnt, 1.0)
    if keep < _LANES:
        y = jnp.concatenate(
            [y[:, :keep], jnp.zeros((y.shape[0], _LANES - keep), y.dtype)],
            axis=1)
    y_ref[...] = y.astype(y_ref.dtype)


def _combine(parts, cnt_parts, keep, out_dtype=jnp.float32, bn=2000):
    _, n, d = parts[0].shape
    spec = pl.BlockSpec((2, bn, d), lambda i: (0, i, 0))
    return pl.pallas_call(
        functools.partial(_combine_body, keep=keep, nparts=len(parts)),
        grid=(n // bn,),
        in_specs=[spec] * (len(parts) + len(cnt_parts)),
        out_specs=pl.BlockSpec((bn, d), lambda i: (i, 0)),
        out_shape=jax.ShapeDtypeStruct((n, d), out_dtype),
        compiler_params=pltpu.CompilerParams(
            dimension_semantics=("arbitrary",)),
    )(*parts, *cnt_parts)


def _sc_gather_pair(table, dst3d, src3d):
    n, d = table.shape
    n_chunks = dst3d.shape[0]
    e = n_chunks * _CE
    k_iters = -(-n_chunks // _NW)
    mesh = plsc.VectorSubcoreMesh(core_axis_name="c", subcore_axis_name="s")

    def body(table_ref, dst_ref, src_ref, xi_ref, xj_ref,
             idx_i, idx_j, rows_i, rows_j, sem_i, sem_j):
        cc = jax.lax.axis_index("c")
        ss = jax.lax.axis_index("s")
        wid = ss * 2 + cc

        def step(k, carry):
            g = wid + _NW * k

            @pl.when(g < n_chunks)
            def _():
                pltpu.sync_copy(dst_ref.at[g], idx_i)
                pltpu.sync_copy(src_ref.at[g], idx_j)
                cps = []
                for j in range(_CH):
                    cps.append(pltpu.async_copy(
                        table_ref.at[idx_i.at[j]],
                        rows_i.at[pl.ds(j * _IDXW, _IDXW)], sem_i))
                    cps.append(pltpu.async_copy(
                        table_ref.at[idx_j.at[j]],
                        rows_j.at[pl.ds(j * _IDXW, _IDXW)], sem_j))
                for cp in cps:
                    cp.wait()
                pltpu.sync_copy(rows_i, xi_ref.at[pl.ds(g * _CE, _CE)])
                pltpu.sync_copy(rows_j, xj_ref.at[pl.ds(g * _CE, _CE)])
            return carry

        jax.lax.fori_loop(0, k_iters, step, 0)

    f = pl.kernel(
        body,
        out_type=[jax.ShapeDtypeStruct((e, d), table.dtype),
                  jax.ShapeDtypeStruct((e, d), table.dtype)],
        mesh=mesh,
        scratch_types=[
            pltpu.VMEM((_CH, _IDXW), jnp.int32),
            pltpu.VMEM((_CH, _IDXW), jnp.int32),
            pltpu.VMEM((_CE, d), table.dtype),
            pltpu.VMEM((_CE, d), table.dtype),
            pltpu.SemaphoreType.DMA,
            pltpu.SemaphoreType.DMA,
        ],
    )
    return f(table, dst3d, src3d)


def _sc_scatter(msg, dst3d, n):
    e, d = msg.shape
    n_chunks = e // _CE
    k_iters = -(-n_chunks // _NW)
    rpt = (n // 16) // 8 * 8
    tail = n - 16 * rpt
    zch = 208
    assert rpt % zch == 0 and tail <= zch
    mesh = plsc.VectorSubcoreMesh(core_axis_name="c", subcore_axis_name="s")

    def body(msg_ref, dst_ref, z_ref, part_ref, idx_v, rows_v, accum):
        cc = jax.lax.axis_index("c")
        ss = jax.lax.axis_index("s")
        wid = ss * 2 + cc

        def striped(fn):
            for i in range(rpt // zch):
                fn(ss * rpt + i * zch, zch)
            if tail:
                @pl.when(ss == 15)
                def _():
                    fn(16 * rpt, tail)

        pltpu.sync_copy(z_ref, rows_v.at[pl.ds(0, zch)])
        striped(lambda at, ln: pltpu.sync_copy(
            rows_v.at[pl.ds(0, ln)], accum.at[pl.ds(at, ln)]))
        plsc.subcore_barrier()

        def step(k, carry):
            g = wid + _NW * k

            @pl.when(g < n_chunks)
            def _():
                pltpu.sync_copy(dst_ref.at[g], idx_v)
                pltpu.sync_copy(msg_ref.at[pl.ds(g * _CE, _CE)], rows_v)
                for j in range(_CH):
                    pltpu.sync_copy(rows_v.at[pl.ds(j * _IDXW, _IDXW)],
                                    accum.at[idx_v.at[j]], add=True)
            return carry

        jax.lax.fori_loop(0, k_iters, step, 0)
        plsc.subcore_barrier()

        def wb(at, ln):
            pltpu.sync_copy(accum.at[pl.ds(at, ln)], rows_v.at[pl.ds(0, ln)])
            pltpu.sync_copy(rows_v.at[pl.ds(0, ln)],
                            part_ref.at[pl.ds(cc * n + at, ln)])
        striped(wb)

    f = pl.kernel(
        body,
        out_type=[jax.ShapeDtypeStruct((2 * n, d), jnp.float32)],
        mesh=mesh,
        scratch_types=[
            pltpu.VMEM((_CH, _IDXW), jnp.int32),
            pltpu.VMEM((_CE, d), jnp.float32),
            pltpu.VMEM_SHARED((n, d), jnp.float32),
        ],
    )
    (out,) = f(msg, dst3d, jnp.zeros((zch, d), jnp.float32))
    return out.reshape(2, n, d)


def kernel(x, edge_index, bn_gamma, bn_beta, eW1, eb1, eW2, eb2, eW3, eb3,
           dW1, db1, dW2, db2, dW3, db3):
    n, d = x.shape
    hid = eW3.shape[1]
    src = edge_index[0].reshape(-1, _CH, _IDXW)
    dst = edge_index[1].reshape(-1, _CH, _IDXW)

    eW1p = jnp.concatenate([eW1[:d] - eW1[d:], eW1[d:]], axis=0)
    dW1p = jnp.concatenate([dW1[:hid] - dW1[hid:], dW1[hid:]], axis=0)

    z = _batchnorm(x, bn_gamma, bn_beta)

    npieces = 5
    nh = dst.shape[0] // npieces
    pieces = [(dst[i * nh:(i + 1) * nh], src[i * nh:(i + 1) * nh])
              for i in range(npieces)]

    def conv(table, w1, b1, w2, b2, w3, b3, last_relu):
        parts = []
        for dh, sh in pieces:
            xi, xj = _sc_gather_pair(table, dh, sh)
            m = _edge_mlp(xi, xj, w1, b1, w2, b2, w3, b3,
                          last_relu=last_relu, be=2000)
            parts.append(_sc_scatter(m, dh, n))
        return parts

    p1 = conv(z, eW1p, eb1, eW2, eb2, eW3, eb3, True)
    y = _combine(p1, p1, keep=hid)
    p2 = conv(y, dW1p, db1, dW2, db2, dW3, db3, False)
    return _combine(p2, p1, keep=d)


# speedup vs baseline: 1.1557x; 1.1557x over previous
import functools

import jax
import jax.numpy as jnp
from jax.experimental import pallas as pl
from jax.experimental.pallas import tpu as pltpu
from jax.experimental.pallas import tpu_sc as plsc

_EPS = 1e-5
_NW = 32
_IDXW = 128
_CH = 2
_CE = _CH * _IDXW
_LANES = 128


def _bn_body(x_ref, g_ref, b_ref, z_ref):
    x = x_ref[...]
    n = x.shape[0]
    mu = jnp.sum(x, axis=0, keepdims=True) / n
    xc = x - mu
    var = jnp.sum(xc * xc, axis=0, keepdims=True) / n
    z_ref[...] = g_ref[...] * xc * jax.lax.rsqrt(var + _EPS) + b_ref[...]


def _batchnorm(x, gamma, beta):
    n, d = x.shape
    return pl.pallas_call(
        _bn_body,
        out_shape=jax.ShapeDtypeStruct((n, d), jnp.float32),
    )(x, gamma.reshape(1, d), beta.reshape(1, d))


def _mlp_body(xi_ref, xj_ref, w1_ref, b1_ref, w2_ref, b2_ref, w3_ref, b3_ref,
              o_ref, *, last_relu):
    din = w1_ref.shape[0] // 2
    dout = w3_ref.shape[1]
    be = xi_ref.shape[0]

    def mm(a, w):
        if a.dtype != jnp.bfloat16:
            a = a.astype(jnp.bfloat16)
        return jnp.dot(a, w, preferred_element_type=jnp.float32)

    h = jnp.concatenate([xi_ref[...][:, :din], xj_ref[...][:, :din]], axis=1)
    h = jnp.maximum(mm(h, w1_ref[...]) + b1_ref[...], 0.0)
    h = jnp.maximum(mm(h, w2_ref[...]) + b2_ref[...], 0.0)
    o = mm(h, w3_ref[...]) + b3_ref[...]
    if last_relu:
        o = jnp.maximum(o, 0.0)
    if dout < _LANES:
        o = jnp.concatenate(
            [o, jnp.full((be, 8), 1.0, jnp.float32),
             jnp.zeros((be, _LANES - dout - 8), jnp.float32)], axis=1)
    o_ref[...] = o


def _edge_mlp(xi, xj, w1, b1, w2, b2, w3, b3, last_relu, be=2048):
    e, din = xi.shape
    big = w1.shape[1]
    dout = w3.shape[1]
    w1 = w1.astype(jnp.bfloat16)
    w2 = w2.astype(jnp.bfloat16)
    w3 = w3.astype(jnp.bfloat16)
    b1 = b1.reshape(1, big)
    b2 = b2.reshape(1, big)
    b3 = b3.reshape(1, dout)
    grid = e // be
    wspec = lambda a: pl.BlockSpec(a.shape, lambda i: (0, 0))
    return pl.pallas_call(
        functools.partial(_mlp_body, last_relu=last_relu),
        grid=(grid,),
        in_specs=[
            pl.BlockSpec((be, din), lambda i: (i, 0)),
            pl.BlockSpec((be, din), lambda i: (i, 0)),
            wspec(w1), wspec(b1), wspec(w2), wspec(b2), wspec(w3), wspec(b3),
        ],
        out_specs=pl.BlockSpec((be, _LANES), lambda i: (i, 0)),
        out_shape=jax.ShapeDtypeStruct((e, _LANES), jnp.float32),
        compiler_params=pltpu.CompilerParams(
            dimension_semantics=("arbitrary",)),
    )(xi, xj, w1, b1, w2, b2, w3, b3)


def _combine_body(*refs, keep, nparts):
    y_ref = refs[-1]
    psum = refs[0][...][0] + refs[0][...][1]
    for r in refs[1:nparts]:
        psum = psum + r[...][0] + r[...][1]
    c = refs[nparts][...][0] + refs[nparts][...][1]
    for r in refs[nparts + 1:2 * nparts]:
        c = c + r[...][0] + r[...][1]
    cnt = c[:, 64:65]
    y = psum / jnp.maximum(cnt, 1.0)
    if keep < _LANES:
        y = jnp.concatenate(
            [y[:, :keep], jnp.zeros((y.shape[0], _LANES - keep), y.dtype)],
            axis=1)
    y_ref[...] = y.astype(y_ref.dtype)


def _combine(parts, cnt_parts, keep, out_dtype=jnp.float32, bn=2000):
    _, n, d = parts[0].shape
    spec = pl.BlockSpec((2, bn, d), lambda i: (0, i, 0))
    return pl.pallas_call(
        functools.partial(_combine_body, keep=keep, nparts=len(parts)),
        grid=(n // bn,),
        in_specs=[spec] * (len(parts) + len(cnt_parts)),
        out_specs=pl.BlockSpec((bn, d), lambda i: (i, 0)),
        out_shape=jax.ShapeDtypeStruct((n, d), out_dtype),
        compiler_params=pltpu.CompilerParams(
            dimension_semantics=("arbitrary",)),
    )(*parts, *cnt_parts)


def _sc_gather_pair(table, dst2d, src2d):
    n, d = table.shape
    n_chunks = dst2d.shape[0]
    e = n_chunks * _IDXW
    k_iters = -(-n_chunks // _NW)
    rpt = (n // 16) // 8 * 8
    tail = n - 16 * rpt
    zch = 104
    assert rpt % zch == 0 and tail <= zch
    mesh = plsc.VectorSubcoreMesh(core_axis_name="c", subcore_axis_name="s")

    def body(table_ref, dst_ref, src_ref, xi_ref, xj_ref,
             idx_i, idx_j, rows_i, rows_j, tab, sem_i, sem_j):
        cc = jax.lax.axis_index("c")
        ss = jax.lax.axis_index("s")
        wid = ss * 2 + cc

        def piece(at, ln):
            pltpu.sync_copy(table_ref.at[pl.ds(at, ln)],
                            rows_i.at[pl.ds(0, ln)])
            pltpu.sync_copy(rows_i.at[pl.ds(0, ln)], tab.at[pl.ds(at, ln)])

        for i in range(rpt // zch):
            piece(ss * rpt + i * zch, zch)
        if tail:
            @pl.when(ss == 15)
            def _():
                piece(16 * rpt, tail)
        plsc.subcore_barrier()

        def step(k, carry):
            g = wid + _NW * k

            @pl.when(g < n_chunks)
            def _():
                pltpu.sync_copy(dst_ref.at[g], idx_i)
                pltpu.sync_copy(src_ref.at[g], idx_j)
                cp_i = pltpu.async_copy(tab.at[idx_i], rows_i, sem_i)
                cp_j = pltpu.async_copy(tab.at[idx_j], rows_j, sem_j)
                cp_i.wait()
                cp_j.wait()
                pltpu.sync_copy(rows_i, xi_ref.at[pl.ds(g * _IDXW, _IDXW)])
                pltpu.sync_copy(rows_j, xj_ref.at[pl.ds(g * _IDXW, _IDXW)])
            return carry

        jax.lax.fori_loop(0, k_iters, step, 0)

    f = pl.kernel(
        body,
        out_type=[jax.ShapeDtypeStruct((e, d), table.dtype),
                  jax.ShapeDtypeStruct((e, d), table.dtype)],
        mesh=mesh,
        scratch_types=[
            pltpu.VMEM((_IDXW,), jnp.int32),
            pltpu.VMEM((_IDXW,), jnp.int32),
            pltpu.VMEM((_IDXW, d), table.dtype),
            pltpu.VMEM((_IDXW, d), table.dtype),
            pltpu.VMEM_SHARED((n, d), table.dtype),
            pltpu.SemaphoreType.DMA,
            pltpu.SemaphoreType.DMA,
        ],
    )
    return f(table, dst2d, src2d)


def _sc_scatter(msg, dst3d, n):
    e, d = msg.shape
    n_chunks = e // _CE
    k_iters = -(-n_chunks // _NW)
    rpt = (n // 16) // 8 * 8
    tail = n - 16 * rpt
    zch = 208
    assert rpt % zch == 0 and tail <= zch
    mesh = plsc.VectorSubcoreMesh(core_axis_name="c", subcore_axis_name="s")

    def body(msg_ref, dst_ref, z_ref, part_ref, idx_v, rows_v, accum):
        cc = jax.lax.axis_index("c")
        ss = jax.lax.axis_index("s")
        wid = ss * 2 + cc

        def striped(fn):
            for i in range(rpt // zch):
                fn(ss * rpt + i * zch, zch)
            if tail:
                @pl.when(ss == 15)
                def _():
                    fn(16 * rpt, tail)

        pltpu.sync_copy(z_ref, rows_v.at[pl.ds(0, zch)])
        striped(lambda at, ln: pltpu.sync_copy(
            rows_v.at[pl.ds(0, ln)], accum.at[pl.ds(at, ln)]))
        plsc.subcore_barrier()

        def step(k, carry):
            g = wid + _NW * k

            @pl.when(g < n_chunks)
            def _():
                pltpu.sync_copy(dst_ref.at[g], idx_v)
                pltpu.sync_copy(msg_ref.at[pl.ds(g * _CE, _CE)], rows_v)
                for j in range(_CH):
                    pltpu.sync_copy(rows_v.at[pl.ds(j * _IDXW, _IDXW)],
                                    accum.at[idx_v.at[j]], add=True)
            return carry

        jax.lax.fori_loop(0, k_iters, step, 0)
        plsc.subcore_barrier()

        def wb(at, ln):
            pltpu.sync_copy(accum.at[pl.ds(at, ln)], rows_v.at[pl.ds(0, ln)])
            pltpu.sync_copy(rows_v.at[pl.ds(0, ln)],
                            part_ref.at[pl.ds(cc * n + at, ln)])
        striped(wb)

    f = pl.kernel(
        body,
        out_type=[jax.ShapeDtypeStruct((2 * n, d), jnp.float32)],
        mesh=mesh,
        scratch_types=[
            pltpu.VMEM((_CH, _IDXW), jnp.int32),
            pltpu.VMEM((_CE, d), jnp.float32),
            pltpu.VMEM_SHARED((n, d), jnp.float32),
        ],
    )
    (out,) = f(msg, dst3d, jnp.zeros((zch, d), jnp.float32))
    return out.reshape(2, n, d)


def kernel(x, edge_index, bn_gamma, bn_beta, eW1, eb1, eW2, eb2, eW3, eb3,
           dW1, db1, dW2, db2, dW3, db3):
    n, d = x.shape
    hid = eW3.shape[1]
    src2 = edge_index[0].reshape(-1, _IDXW)
    dst2 = edge_index[1].reshape(-1, _IDXW)
    src = edge_index[0].reshape(-1, _CH, _IDXW)
    dst = edge_index[1].reshape(-1, _CH, _IDXW)

    eW1p = jnp.concatenate([eW1[:d] - eW1[d:], eW1[d:]], axis=0)
    dW1p = jnp.concatenate([dW1[:hid] - dW1[hid:], dW1[hid:]], axis=0)

    z = _batchnorm(x, bn_gamma, bn_beta)

    npieces = 2
    nh = dst.shape[0] // npieces
    nh2 = dst2.shape[0] // npieces
    pieces = [(dst[i * nh:(i + 1) * nh],
               dst2[i * nh2:(i + 1) * nh2],
               src2[i * nh2:(i + 1) * nh2]) for i in range(npieces)]

    def conv(table, w1, b1, w2, b2, w3, b3, last_relu):
        parts = []
        for dh, dh2, sh2 in pieces:
            xi, xj = _sc_gather_pair(table, dh2, sh2)
            m = _edge_mlp(xi, xj, w1, b1, w2, b2, w3, b3,
                          last_relu=last_relu, be=2000)
            parts.append(_sc_scatter(m, dh, n))
        return parts

    p1 = conv(z, eW1p, eb1, eW2, eb2, eW3, eb3, True)
    y = _combine(p1, p1, keep=hid)
    p2 = conv(y, dW1p, db1, dW2, db2, dW3, db3, False)
    return _combine(p2, p1, keep=d)
